# baseline (speedup 1.0000x reference)
import functools

import numpy as np

import jax
import jax.numpy as jnp
from jax import lax
from jax.experimental import pallas as pl
from jax.experimental.pallas import tpu as pltpu
from jax.experimental.pallas import tpu_sc as plsc

NORM_TERM = 13.0
KNOISE = 100
P = 112
T = 1024
D = 64
DP = D // 2
NW = 32
NS = 16
TW = T // NW
VOCAB = 100000
VHALF = VOCAB // 2
LN2 = 0.6931471805599453
NBPAD = 6272


def _rotl(x, r):
    return ((x << np.uint32(r)) | (x >> np.uint32(32 - r))).astype(np.uint32)


def _threefry2x32(k0, k1, x0, x1):
    rotations = ((13, 15, 26, 6), (17, 29, 16, 24))
    ks = (np.uint32(k0), np.uint32(k1),
          np.uint32(k0) ^ np.uint32(k1) ^ np.uint32(0x1BD11BDA))
    x0 = (x0 + ks[0]).astype(np.uint32)
    x1 = (x1 + ks[1]).astype(np.uint32)
    for i in range(5):
        for r in rotations[i % 2]:
            x0 = (x0 + x1).astype(np.uint32)
            x1 = _rotl(x1, r)
            x1 = x1 ^ x0
        x0 = (x0 + ks[(i + 1) % 3]).astype(np.uint32)
        x1 = (x1 + ks[(i + 2) % 3] + np.uint32(i + 1)).astype(np.uint32)
    return x0, x1


def _random_bits_32(k0, k1, size):
    idx = np.arange(size, dtype=np.uint64)
    c1 = (idx >> np.uint64(32)).astype(np.uint32)
    c2 = (idx & np.uint64(0xFFFFFFFF)).astype(np.uint32)
    b1, b2 = _threefry2x32(k0, k1, c1, c2)
    return b1 ^ b2


def _np_randint(seed, size, span):
    k0, k1 = np.uint32(0), np.uint32(seed)
    c1 = np.zeros(2, np.uint32)
    c2 = np.arange(2, dtype=np.uint32)
    b1, b2 = _threefry2x32(k0, k1, c1, c2)
    higher = _random_bits_32(b1[0], b2[0], size)
    lower = _random_bits_32(b1[1], b2[1], size)
    span = np.uint32(span)
    mult = np.uint32(2 ** 16) % span
    with np.errstate(over="ignore"):
        mult = np.uint32(mult * mult) % span
        off = ((higher % span) * mult + lower % span) % span
    return off.astype(np.int32)


def _np_uniform(seed, size):
    k0, k1 = np.uint32(0), np.uint32(seed)
    bits = _random_bits_32(k0, k1, size)
    fb = (bits >> np.uint32(9)) | np.uint32(0x3F800000)
    return fb.view(np.float32) - np.float32(1.0)


_CONSTS = None


def _prng_consts():
    global _CONSTS
    if _CONSTS is None:
        kk2 = np.zeros((T, P), np.int32)
        kk2[:, :KNOISE] = _np_randint(42, T * KNOISE, VOCAB).reshape(T, KNOISE)
        u2 = np.full((T, P), 2.0, np.float32)
        u2[:, :KNOISE] = _np_uniform(43, T * KNOISE).reshape(T, KNOISE)
        _CONSTS = (kk2, u2)
    return _CONSTS


def _ln(x):
    xi = plsc.bitcast(x, jnp.int32)
    e = ((xi >> 23) & 0xFF) - 127
    m = plsc.bitcast((xi & 0x007FFFFF) | 0x3F800000, jnp.float32)
    z = (m - 1.0) / (m + 1.0)
    z2 = z * z
    lnm = 2.0 * z * (1.0 + z2 * (0.33333333333 + z2 * (0.2 + z2 * 0.14285714285)))
    return e.astype(jnp.float32) * LN2 + lnm


def _sc_body(kk_hbm, u_hbm, tgt_hbm, embe_hbm, embo_hbm, t1_hbm, nb_hbm,
             w2_hbm, out_hbm,
             kk_tok, u_tok, ns_all, nb_tok, rows, embe_v, embo_v,
             tv, trows, tnb, idx_v, scores_v, lacc,
             sh_w, sh_nb,
             semw, semt1, semnb):
    c = lax.axis_index("c")
    s = lax.axis_index("s")
    wid = c * NS + s
    base = wid * TW
    T1R = 6400 // NS

    cp_t1 = pltpu.make_async_copy(t1_hbm.at[pl.ds(s * T1R, T1R)],
                                  sh_w.at[pl.ds(s * T1R, T1R)], semt1)
    cp_nb = pltpu.make_async_copy(nb_hbm.at[pl.ds(s * NBPAD, NBPAD)],
                                  sh_nb.at[pl.ds(s * NBPAD, NBPAD)], semnb)
    cp_t1.start()
    cp_nb.start()

    pltpu.sync_copy(embe_hbm.at[pl.ds(base, TW)], embe_v)
    pltpu.sync_copy(embo_hbm.at[pl.ds(base, TW)], embo_v)
    pltpu.sync_copy(tgt_hbm.at[pl.ds(base, TW)], tv)

    iota16 = lax.iota(jnp.int32, 16)
    sidx = [iota16 + (g * 16) for g in range(P // 16)]

    def _pidx(dstep):
        v = iota16 + dstep
        return jnp.where(v >= DP, v - DP, v)

    lane_valid = [jnp.full((16,), True)] * 6 + [iota16 < 4]

    lacc[pl.ds(0, 16)] = jnp.zeros((16,), jnp.float32)

    cp_t1.wait()
    cp_nb.wait()
    plsc.subcore_barrier()

    def phase_a(lt, carry):
        pltpu.sync_copy(kk_hbm.at[base + lt], kk_tok)
        pltpu.sync_copy(u_hbm.at[base + lt], u_tok)
        for g in range(P // 16):
            sl = pl.ds(g * 16, 16)
            idx_v[sl] = kk_tok[sl] >> 4
        pltpu.sync_copy(sh_w.at[idx_v], rows)
        for g in range(P // 16):
            sl = pl.ds(g * 16, 16)
            kkg = kk_tok[sl]
            col = (kkg & 15) << 1
            ap = plsc.bitcast(plsc.load_gather(rows, [sidx[g], col]),
                              jnp.float32)
            aa = plsc.load_gather(rows, [sidx[g], col + 1])
            bsel = u_tok[sl] < ap
            ns_all[lt, sl] = jnp.where(bsel, kkg, aa)
        return carry

    lax.fori_loop(0, TW, phase_a, 0)
    pltpu.sync_copy(sh_nb.at[tv], tnb)

    def _unpack(w):
        wlo = plsc.bitcast(w << 16, jnp.float32)
        whi = plsc.bitcast(w & jnp.int32(-65536), jnp.float32)
        return wlo, whi

    def one_pass(h, carry):
        plsc.subcore_barrier()
        pltpu.sync_copy(
            w2_hbm.at[pl.ds(h * VHALF + s * (VHALF // NS), VHALF // NS)],
            sh_w.at[pl.ds(s * (VHALF // NS), VHALF // NS)])
        plsc.subcore_barrier()
        lo = h * VHALF

        def token_body(lt, carry2):
            for g in range(P // 16):
                sl = pl.ds(g * 16, 16)
                loc = ns_all[lt, sl] - lo
                selg = (loc >= 0) & (loc < VHALF)
                idx_v[sl] = jnp.where(selg, loc, 0)
            pltpu.sync_copy(sh_w.at[idx_v], rows)
            pltpu.sync_copy(sh_nb.at[ns_all.at[lt]], nb_tok)

            for g0 in range(0, P // 16, 2):
                gs = list(range(g0, min(g0 + 2, P // 16)))

                def dot_step(dstep, ch, gs=gs):
                    ee = embe_v[lt, pl.ds(dstep, 16)]
                    eo = embo_v[lt, pl.ds(dstep, 16)]
                    pidx = _pidx(dstep)
                    out = []
                    for i, g in enumerate(gs):
                        w = plsc.load_gather(rows, [sidx[g], pidx])
                        wlo, whi = _unpack(w)
                        out.append(ch[i] + wlo * ee + whi * eo)
                    return tuple(out)

                chunk = lax.fori_loop(
                    0, DP, dot_step,
                    tuple(jnp.zeros((16,), jnp.float32) for _ in gs))
                for i, g in enumerate(gs):
                    scores_v[pl.ds(g * 16, 16)] = chunk[i]

            bce_sum = jnp.zeros((16,), jnp.float32)
            for g in range(P // 16):
                sl = pl.ds(g * 16, 16)
                loc = ns_all[lt, sl] - lo
                selg = (loc >= 0) & (loc < VHALF)
                biasg, noiseg = _unpack(nb_tok[sl])
                score = scores_v[sl] + biasg
                pm = jnp.clip(jnp.exp(score - NORM_TERM), 1e-9, 1.0)
                p = pm / (pm + 100.0 * noiseg)
                p = jnp.clip(p, 1e-12, 1.0 - 1e-12)
                bce = -_ln(1.0 - p)
                bce_sum = bce_sum + jnp.where(selg & lane_valid[g], bce, 0.0)
            lacc[pl.ds(0, 16)] = lacc[pl.ds(0, 16)] + bce_sum
            return carry2

        lax.fori_loop(0, TW, token_body, 0)

        for tg in range(TW // 16):
            sl = pl.ds(tg * 16, 16)
            tok16 = iota16 + (tg * 16)
            tvv = tv[sl]
            loct = tvv - lo
            selt = (loct >= 0) & (loct < VHALF)
            idx_v[pl.ds(0, 16)] = jnp.where(selt, loct, 0)
            pltpu.sync_copy(sh_w.at[idx_v.at[pl.ds(0, 16)]],
                            trows.at[sl])
            def t_step(dstep, acc, tok16=tok16):
                pidx = _pidx(dstep)
                w = plsc.load_gather(trows, [tok16, pidx])
                wlo, whi = _unpack(w)
                ee = plsc.load_gather(embe_v, [tok16, pidx])
                eo = plsc.load_gather(embo_v, [tok16, pidx])
                return acc + wlo * ee + whi * eo

            acc = lax.fori_loop(0, DP, t_step,
                                jnp.zeros((16,), jnp.float32))
            tb, tn = _unpack(tnb[sl])
            score_t = acc + tb
            pmt = jnp.clip(jnp.exp(score_t - NORM_TERM), 1e-9, 1.0)
            pt = pmt / (pmt + 100.0 * tn)
            pt = jnp.clip(pt, 1e-12, 1.0 - 1e-12)
            bce_t = -_ln(pt)
            lacc[pl.ds(0, 16)] = (lacc[pl.ds(0, 16)] +
                                  jnp.where(selt, bce_t, 0.0))
        return carry

    lax.fori_loop(0, 2, one_pass, 0)

    pltpu.sync_copy(lacc, out_hbm.at[wid])


_sc_call = functools.partial(
    pl.kernel,
    out_type=[
        jax.ShapeDtypeStruct((NW, 16), jnp.float32),
    ],
    mesh=plsc.VectorSubcoreMesh(core_axis_name="c", subcore_axis_name="s"),
    compiler_params=pltpu.CompilerParams(use_tc_tiling_on_sc=False,
                                         needs_layout_passes=False),
    scratch_types=[
        pltpu.VMEM((P,), jnp.int32),
        pltpu.VMEM((P,), jnp.float32),
        pltpu.VMEM((TW, P), jnp.int32),
        pltpu.VMEM((P,), jnp.int32),
        pltpu.VMEM((P, DP), jnp.int32),
        pltpu.VMEM((TW, D), jnp.float32),
        pltpu.VMEM((TW, D), jnp.float32),
        pltpu.VMEM((TW,), jnp.int32),
        pltpu.VMEM((TW, DP), jnp.int32),
        pltpu.VMEM((TW,), jnp.int32),
        pltpu.VMEM((P,), jnp.int32),
        pltpu.VMEM((P,), jnp.float32),
        pltpu.VMEM((16,), jnp.float32),
        pltpu.VMEM_SHARED((VHALF, DP), jnp.int32),
        pltpu.VMEM_SHARED((NBPAD * NS,), jnp.int32),
        pltpu.SemaphoreType.DMA,
        pltpu.SemaphoreType.DMA,
        pltpu.SemaphoreType.DMA,
    ],
)(_sc_body)


def _tc_body(part_ref, out_ref):
    out_ref[0, 0] = jnp.sum(part_ref[...]) / float(T)


def _tc_call(parts):
    return pl.pallas_call(
        _tc_body,
        out_shape=jax.ShapeDtypeStruct((1, 1), jnp.float32),
        out_specs=pl.BlockSpec(memory_space=pltpu.SMEM),
    )(parts)


def kernel(target, emb, noise, alias_prob, alias_alias, weight, bias):
    tgt = target.reshape(T).astype(jnp.int32)
    embf = emb.reshape(T, D)
    aa = alias_alias.astype(jnp.int32)
    kk2np, u2np = _prng_consts()
    kk2 = jnp.asarray(kk2np)
    u2 = jnp.asarray(u2np)
    wbf = weight.astype(jnp.bfloat16).reshape(VOCAB, DP, 2)
    w2 = lax.bitcast_convert_type(wbf, jnp.int32)
    t1 = jnp.stack([lax.bitcast_convert_type(alias_prob, jnp.int32), aa],
                   axis=1)
    t1p = jnp.pad(t1.reshape(VOCAB // 16, 32), ((0, 150), (0, 0)))
    nbp = lax.bitcast_convert_type(
        jnp.stack([bias, noise], axis=1).astype(jnp.bfloat16),
        jnp.int32)
    nbp = jnp.pad(nbp, (0, NBPAD * NS - VOCAB))
    ep = embf.reshape(T, DP, 2)
    ee = ep[:, :, 0]
    eo = ep[:, :, 1]
    embe2 = jnp.concatenate([ee, ee], axis=1)
    embo2 = jnp.concatenate([eo, eo], axis=1)
    parts = _sc_call(kk2, u2, tgt, embe2, embo2, t1p, nbp, w2)
    if isinstance(parts, (list, tuple)):
        parts = parts[0]
    loss = _tc_call(parts)
    return loss[0, 0]
